# Initial kernel scaffold; baseline (speedup 1.0000x reference)
import jax
import jax.numpy as jnp
from jax.experimental import pallas as pl


def kernel(hs, adj, W, a, W1, b1, W2):
    raise NotImplementedError("write your pallas kernel here")


# speedup vs baseline: 1.9019x; 1.9019x over previous
import functools

import jax
import jax.numpy as jnp
from jax.experimental import pallas as pl
from jax.experimental.pallas import tpu as pltpu

M, N, IN, OUT, HID = 3, 4096, 128, 64, 128
ALPHA = 0.2
BR = 256


def _gat_kernel(hs_ref, adj_ref, w_ref, a1_ref, a2_ref, z_ref,
                wh_s, f1_s, f2_s):
    r = pl.program_id(1)

    @pl.when(r == 0)
    def _():
        wh = jnp.dot(hs_ref[0], w_ref[...],
                     preferred_element_type=jnp.float32)
        wh_s[...] = wh
        f1_s[...] = jnp.dot(wh, a1_ref[...],
                            preferred_element_type=jnp.float32)
        f2_s[...] = jax.lax.dot_general(
            a2_ref[...], wh, (((1,), (1,)), ((), ())),
            preferred_element_type=jnp.float32)

    wh = wh_s[...]
    f1b = f1_s[pl.ds(r * BR, BR), :]
    e = f1b + f2_s[...]
    e = jnp.where(e > 0, e, ALPHA * e)
    att = jnp.where(adj_ref[0] > 0, e, jnp.float32(-9e15))
    m = jnp.max(att, axis=1, keepdims=True)
    p = jnp.exp(att - m)
    s = jnp.sum(p, axis=1, keepdims=True)
    att = p / s
    hp = jnp.dot(att, wh, preferred_element_type=jnp.float32)
    z_ref[0] = jnp.where(hp > 0, hp, jnp.exp(jnp.minimum(hp, 0.0)) - 1.0)


def _semantic_kernel(z_ref, w1_ref, b1_ref, w2_ref, out_ref):
    ws = []
    for i in range(M):
        zi = z_ref[i]
        t = jnp.tanh(jnp.dot(zi, w1_ref[...],
                             preferred_element_type=jnp.float32)
                     + b1_ref[...])
        wv = jnp.dot(t, w2_ref[...],
                     preferred_element_type=jnp.float32)
        ws.append(jnp.sum(wv, keepdims=True) / N)
    mx = jnp.maximum(jnp.maximum(ws[0], ws[1]), ws[2])
    es = [jnp.exp(w - mx) for w in ws]
    denom = es[0] + es[1] + es[2]
    out = (es[0] / denom) * z_ref[0]
    out += (es[1] / denom) * z_ref[1]
    out += (es[2] / denom) * z_ref[2]
    out_ref[...] = out


@jax.jit
def kernel(hs, adj, W, a, W1, b1, W2):
    a1 = a[:OUT]
    a2 = a[OUT:].reshape(1, OUT)
    b1r = b1.reshape(1, HID)

    z = pl.pallas_call(
        _gat_kernel,
        grid=(M, N // BR),
        in_specs=[
            pl.BlockSpec((1, N, IN), lambda i, r: (i, 0, 0)),
            pl.BlockSpec((1, BR, N), lambda i, r: (i, r, 0)),
            pl.BlockSpec((IN, OUT), lambda i, r: (0, 0)),
            pl.BlockSpec((OUT, 1), lambda i, r: (0, 0)),
            pl.BlockSpec((1, OUT), lambda i, r: (0, 0)),
        ],
        out_specs=pl.BlockSpec((1, BR, OUT), lambda i, r: (i, r, 0)),
        out_shape=jax.ShapeDtypeStruct((M, N, OUT), jnp.float32),
        scratch_shapes=[
            pltpu.VMEM((N, OUT), jnp.float32),
            pltpu.VMEM((N, 1), jnp.float32),
            pltpu.VMEM((1, N), jnp.float32),
        ],
        compiler_params=pltpu.CompilerParams(
            dimension_semantics=("arbitrary", "arbitrary")),
    )(hs, adj, W, a1, a2)

    out = pl.pallas_call(
        _semantic_kernel,
        in_specs=[
            pl.BlockSpec((M, N, OUT), lambda: (0, 0, 0)),
            pl.BlockSpec((OUT, HID), lambda: (0, 0)),
            pl.BlockSpec((1, HID), lambda: (0, 0)),
            pl.BlockSpec((HID, 1), lambda: (0, 0)),
        ],
        out_specs=pl.BlockSpec((N, OUT), lambda: (0, 0)),
        out_shape=jax.ShapeDtypeStruct((N, OUT), jnp.float32),
    )(z, W1, b1r, W2)
    return out


# speedup vs baseline: 2.4499x; 1.2882x over previous
import jax
import jax.numpy as jnp
from jax.experimental import pallas as pl
from jax.experimental.pallas import tpu as pltpu

M, N, IN, OUT, HID = 3, 4096, 128, 64, 128
ALPHA = 0.2
BR = 256


def _gat_kernel(hs_ref, adj_ref, w_ref, a1_ref, a2_ref, z_ref,
                whb_s, e1_s, g1_s, e2_s, g2_s, cm_s):
    r = pl.program_id(1)

    @pl.when(r == 0)
    def _():
        wh = jnp.dot(hs_ref[0], w_ref[...],
                     preferred_element_type=jnp.float32)
        whb_s[:, :OUT] = wh.astype(jnp.bfloat16)
        whb_s[:, OUT:] = jnp.ones((N, OUT), jnp.bfloat16)
        f1 = jnp.dot(wh, a1_ref[...],
                     preferred_element_type=jnp.float32)
        e1_s[...] = jnp.exp(f1)
        g1_s[...] = jnp.exp(ALPHA * f1)
        f2 = jax.lax.dot_general(
            a2_ref[...], wh, (((1,), (1,)), ((), ())),
            preferred_element_type=jnp.float32)
        e2_s[...] = jnp.exp(f2)
        g2_s[...] = jnp.exp(ALPHA * f2)
        cm_s[...] = jnp.mean(wh, axis=0, keepdims=True)

    e1b = e1_s[pl.ds(r * BR, BR), :]
    g1b = g1_s[pl.ds(r * BR, BR), :]
    p = jnp.maximum(e1b * e2_s[...], g1b * g2_s[...])
    p = jnp.where(adj_ref[0] > 0, p, 0.0).astype(jnp.bfloat16)
    h = jnp.dot(p, whb_s[...],
                preferred_element_type=jnp.float32)
    s = h[:, OUT:OUT + 1]
    hp = h[:, :OUT] / jnp.maximum(s, 1e-30)
    hp = jnp.where(s > 0, hp, cm_s[...])
    z_ref[0] = jnp.where(hp > 0, hp, jnp.exp(jnp.minimum(hp, 0.0)) - 1.0)


def _semantic_kernel(z_ref, w1_ref, b1_ref, w2_ref, out_ref):
    ws = []
    for i in range(M):
        t = jnp.tanh(jnp.dot(z_ref[i], w1_ref[...],
                             preferred_element_type=jnp.float32)
                     + b1_ref[...])
        wv = jnp.dot(t, w2_ref[...],
                     preferred_element_type=jnp.float32)
        ws.append(jnp.sum(wv, keepdims=True) / N)
    mx = jnp.maximum(jnp.maximum(ws[0], ws[1]), ws[2])
    es = [jnp.exp(w - mx) for w in ws]
    denom = es[0] + es[1] + es[2]
    out = (es[0] / denom) * z_ref[0]
    out += (es[1] / denom) * z_ref[1]
    out += (es[2] / denom) * z_ref[2]
    out_ref[...] = out


@jax.jit
def kernel(hs, adj, W, a, W1, b1, W2):
    a1 = a[:OUT]
    a2 = a[OUT:].reshape(1, OUT)
    b1r = b1.reshape(1, HID)

    z = pl.pallas_call(
        _gat_kernel,
        grid=(M, N // BR),
        in_specs=[
            pl.BlockSpec((1, N, IN), lambda i, r: (i, 0, 0)),
            pl.BlockSpec((1, BR, N), lambda i, r: (i, r, 0)),
            pl.BlockSpec((IN, OUT), lambda i, r: (0, 0)),
            pl.BlockSpec((OUT, 1), lambda i, r: (0, 0)),
            pl.BlockSpec((1, OUT), lambda i, r: (0, 0)),
        ],
        out_specs=pl.BlockSpec((1, BR, OUT), lambda i, r: (i, r, 0)),
        out_shape=jax.ShapeDtypeStruct((M, N, OUT), jnp.float32),
        scratch_shapes=[
            pltpu.VMEM((N, 2 * OUT), jnp.bfloat16),
            pltpu.VMEM((N, 1), jnp.float32),
            pltpu.VMEM((N, 1), jnp.float32),
            pltpu.VMEM((1, N), jnp.float32),
            pltpu.VMEM((1, N), jnp.float32),
            pltpu.VMEM((1, OUT), jnp.float32),
        ],
        compiler_params=pltpu.CompilerParams(
            dimension_semantics=("arbitrary", "arbitrary")),
    )(hs, adj, W, a1, a2)

    out = pl.pallas_call(
        _semantic_kernel,
        in_specs=[
            pl.BlockSpec((M, N, OUT), lambda: (0, 0, 0)),
            pl.BlockSpec((OUT, HID), lambda: (0, 0)),
            pl.BlockSpec((1, HID), lambda: (0, 0)),
            pl.BlockSpec((HID, 1), lambda: (0, 0)),
        ],
        out_specs=pl.BlockSpec((N, OUT), lambda: (0, 0)),
        out_shape=jax.ShapeDtypeStruct((N, OUT), jnp.float32),
    )(z, W1, b1r, W2)
    return out
